# baseline (speedup 1.0000x reference)
import functools

import jax
import jax.numpy as jnp
from jax import lax
from jax.experimental import pallas as pl
from jax.experimental.pallas import tpu as pltpu
from jax.experimental.pallas import tpu_sc as plsc


def _tc_body(y_ref, x_ref, loc_ref, W2_ref, bf_ref, W1_ref, w0_ref, c_ref,
             d_ref, ovt_ref):
    xb = x_ref[0]
    xb = jnp.where(jnp.isnan(xb), 0.0, xb)
    xa = jnp.dot(xb, W2_ref[...], preferred_element_type=jnp.float32)
    xa = xa + bf_ref[...]
    k = jax.lax.broadcasted_iota(jnp.int32, xa.shape, 1) % 6
    feat = jnp.where(k == 0, xa, jnp.sin(xa))
    tp = jnp.dot(feat, W1_ref[...], preferred_element_type=jnp.float32)
    base = loc_ref[...] + tp + c_ref[...]
    yb = y_ref[0]
    nanm = jnp.isnan(yb)
    y0 = jnp.where(nanm, 0.0, yb)
    nf = nanm.astype(jnp.float32)
    w0 = w0_ref[...]
    dl = d_ref[...]
    for i in range(8):
        ovt_ref[0, i] = base + y0[:, i:i + 1] * w0 + nf[:, i:i + 1] * dl


def _val_time_tc(y, x, local_emb_table, W2, b_flat, vt_W1, w0row, crow, drow):
    bs, L, d_y = y.shape
    d_x = x.shape[-1]
    d_model = local_emb_table.shape[-1]
    TD = W2.shape[1]
    TB = 256
    nt = L // TB
    vt4 = pl.pallas_call(
        _tc_body,
        grid=(bs, nt),
        in_specs=[
            pl.BlockSpec((1, TB, d_y), lambda b, t: (b, t, 0)),
            pl.BlockSpec((1, TB, d_x), lambda b, t: (b, t, 0)),
            pl.BlockSpec((TB, d_model), lambda b, t: (t, 0)),
            pl.BlockSpec((d_x, TD), lambda b, t: (0, 0)),
            pl.BlockSpec((1, TD), lambda b, t: (0, 0)),
            pl.BlockSpec((TD, d_model), lambda b, t: (0, 0)),
            pl.BlockSpec((1, d_model), lambda b, t: (0, 0)),
            pl.BlockSpec((1, d_model), lambda b, t: (0, 0)),
            pl.BlockSpec((1, d_model), lambda b, t: (0, 0)),
        ],
        out_specs=pl.BlockSpec((1, d_y, TB, d_model),
                               lambda b, t: (b, 0, t, 0)),
        out_shape=jax.ShapeDtypeStruct((bs, d_y, L, d_model), jnp.float32),
        compiler_params=pltpu.CompilerParams(
            dimension_semantics=("parallel", "parallel")),
    )(y, x, local_emb_table, W2, b_flat, vt_W1, w0row, crow, drow)
    return vt4.reshape(bs, d_y * L, d_model)


def _space_sc(space_table, bs, L, d_model):
    d_y = space_table.shape[0]
    info = plsc.get_sparse_core_info()
    NW = info.num_cores * info.num_subcores
    rows = bs * d_y * L // NW
    K = 128
    mesh = plsc.VectorSubcoreMesh(core_axis_name="c", subcore_axis_name="s")

    @functools.partial(
        pl.kernel, mesh=mesh,
        out_type=jax.ShapeDtypeStruct((NW, rows, d_model), jnp.float32),
        scratch_types=[
            pltpu.VMEM((K,), jnp.int32),
            pltpu.VMEM((K, d_model), jnp.float32),
            pltpu.SemaphoreType.DMA,
        ],
    )
    def k(table_hbm, out_hbm, idx_v, buf, sem):
        wid = lax.axis_index("s") * info.num_cores + lax.axis_index("c")
        tid = wid % d_y
        splat = jnp.full((16,), tid, dtype=jnp.int32)
        for c in range(K // 16):
            idx_v[pl.ds(c * 16, 16)] = splat
        pltpu.async_copy(table_hbm.at[idx_v], buf, sem).wait()
        copies = [
            pltpu.async_copy(buf, out_hbm.at[wid, pl.ds(j * K, K)], sem)
            for j in range(rows // K)
        ]
        for c in copies:
            c.wait()

    sp = k(space_table)
    return sp.reshape(bs, d_y * L, d_model)


def kernel(y, x, local_emb_table, time_w, time_b, vt_W, vt_b, space_table,
           given_table):
    bs, L, d_y = y.shape
    d_x = x.shape[-1]
    d_model = local_emb_table.shape[-1]
    E = time_w.shape[1]
    TD = d_x * E

    W2 = (jnp.eye(d_x, dtype=jnp.float32)[:, :, None]
          * time_w[None, :, :]).reshape(d_x, TD)
    b_flat = time_b.reshape(1, TD)
    vt_W1 = vt_W[1:]
    w0row = vt_W[0:1]
    crow = (vt_b + given_table[1])[None, :]
    drow = (given_table[0] - given_table[1])[None, :]

    sp = _space_sc(space_table, bs, L, d_model)
    vt = _val_time_tc(y, x, local_emb_table, W2, b_flat, vt_W1, w0row, crow,
                      drow)
    return (vt, sp)
